# Initial kernel scaffold; baseline (speedup 1.0000x reference)
import jax
import jax.numpy as jnp
from jax.experimental import pallas as pl


def kernel(title, table):
    raise NotImplementedError("write your pallas kernel here")


# speedup vs baseline: 1.0936x; 1.0936x over previous
import functools

import jax
import jax.numpy as jnp
from jax import lax
from jax.experimental import pallas as pl
from jax.experimental.pallas import tpu as pltpu
from jax.experimental.pallas import tpu_sc as plsc

VOCAB = 1000000
EMBED_DIM = 32
BATCH = 16384
HIST_LEN = 50
TOTAL = BATCH * HIST_LEN

NC = 2
NS = 16
NW = NC * NS
B_PER_W = TOTAL // NW

GROUP = 128
K = 8
CHUNK = K * GROUP
NCHUNKS = B_PER_W // CHUNK

_mesh = plsc.VectorSubcoreMesh(core_axis_name="c", subcore_axis_name="s")


@functools.partial(
    pl.kernel,
    out_type=jax.ShapeDtypeStruct((TOTAL, EMBED_DIM), jnp.float32),
    mesh=_mesh,
    scratch_types=[
        pltpu.VMEM((K, GROUP), jnp.int32),
        pltpu.VMEM((CHUNK, EMBED_DIM), jnp.float32),
        pltpu.SemaphoreType.DMA,
    ],
    compiler_params=pltpu.CompilerParams(use_tc_tiling_on_sc=False),
)
def _emb_gather(idx_hbm, table_hbm, out_hbm, idx_v, rows_v, sem):
    wid = lax.axis_index("s") * NC + lax.axis_index("c")
    base = wid * B_PER_W

    def chunk_body(c, carry):
        row_off = pl.multiple_of(base + c * CHUNK, CHUNK)
        grp_off = pl.multiple_of(row_off // GROUP, K)
        pltpu.sync_copy(idx_hbm.at[pl.ds(grp_off, K)], idx_v)
        copies = []
        for j in range(K):
            copies.append(
                pltpu.async_copy(
                    table_hbm.at[idx_v.at[j]],
                    rows_v.at[pl.ds(j * GROUP, GROUP)],
                    sem,
                )
            )
        for cp in copies:
            cp.wait()
        pltpu.sync_copy(rows_v, out_hbm.at[pl.ds(row_off, CHUNK)])
        return carry

    lax.fori_loop(0, NCHUNKS, chunk_body, 0)


def kernel(title, table):
    idx = title.astype(jnp.int32).reshape(TOTAL // GROUP, GROUP)
    out = _emb_gather(idx, table)
    return out.reshape(BATCH, HIST_LEN, EMBED_DIM)


# speedup vs baseline: 1.1125x; 1.0173x over previous
import functools

import jax
import jax.numpy as jnp
from jax import lax
from jax.experimental import pallas as pl
from jax.experimental.pallas import tpu as pltpu
from jax.experimental.pallas import tpu_sc as plsc

VOCAB = 1000000
EMBED_DIM = 32
BATCH = 16384
HIST_LEN = 50
TOTAL = BATCH * HIST_LEN

NC = 2
NS = 16
NW = NC * NS
B_PER_W = TOTAL // NW

GROUP = 128
NGRP_W = B_PER_W // GROUP
K = 10
CHUNK = K * GROUP
NCHUNKS = NGRP_W // K
NPAIRS = NCHUNKS // 2

_mesh = plsc.VectorSubcoreMesh(core_axis_name="c", subcore_axis_name="s")


@functools.partial(
    pl.kernel,
    out_type=jax.ShapeDtypeStruct((TOTAL, EMBED_DIM), jnp.float32),
    mesh=_mesh,
    scratch_types=[
        pltpu.VMEM((NGRP_W, GROUP), jnp.int32),
        pltpu.VMEM((CHUNK, EMBED_DIM), jnp.float32),
        pltpu.VMEM((CHUNK, EMBED_DIM), jnp.float32),
        pltpu.SemaphoreType.DMA,
        pltpu.SemaphoreType.DMA,
        pltpu.SemaphoreType.DMA,
        pltpu.SemaphoreType.DMA,
        pltpu.SemaphoreType.DMA,
    ],
    compiler_params=pltpu.CompilerParams(use_tc_tiling_on_sc=False),
)
def _emb_gather(idx_hbm, table_hbm, out_hbm, idx_all, rows0, rows1,
                isem, gsem0, gsem1, wsem0, wsem1):
    wid = lax.axis_index("s") * NC + lax.axis_index("c")
    base = pl.multiple_of(wid * B_PER_W, B_PER_W)
    gbase = pl.multiple_of(wid * NGRP_W, NGRP_W)

    pltpu.async_copy(idx_hbm.at[pl.ds(gbase, NGRP_W)], idx_all, isem).wait()

    rows = (rows0, rows1)
    gsem = (gsem0, gsem1)
    wsem = (wsem0, wsem1)

    def fire_gathers(c, slot):
        for j in range(K):
            pltpu.async_copy(
                table_hbm.at[idx_all.at[c * K + j]],
                rows[slot].at[pl.ds(j * GROUP, GROUP)],
                gsem[slot],
            )

    def drain_gathers(slot):
        pltpu.make_async_copy(
            table_hbm.at[pl.ds(0, CHUNK)], rows[slot], gsem[slot]
        ).wait()

    def fire_write(c, slot):
        pltpu.async_copy(
            rows[slot], out_hbm.at[pl.ds(base + c * CHUNK, CHUNK)], wsem[slot]
        )

    def drain_write(slot):
        pltpu.make_async_copy(
            rows[slot], out_hbm.at[pl.ds(0, CHUNK)], wsem[slot]
        ).wait()

    fire_gathers(0, 0)

    def pair_body(p, carry):
        c0 = p * 2
        c1 = c0 + 1

        @pl.when(p > 0)
        def _():
            drain_write(1)

        fire_gathers(c1, 1)

        drain_gathers(0)
        fire_write(c0, 0)

        @pl.when(p + 1 < NPAIRS)
        def _():
            drain_write(0)
            fire_gathers(c0 + 2, 0)

        drain_gathers(1)
        fire_write(c1, 1)
        return carry

    lax.fori_loop(0, NPAIRS, pair_body, 0)
    drain_write(0)
    drain_write(1)


def kernel(title, table):
    idx = title.astype(jnp.int32).reshape(TOTAL // GROUP, GROUP)
    out = _emb_gather(idx, table)
    return out.reshape(BATCH, HIST_LEN, EMBED_DIM)


# speedup vs baseline: 1.8055x; 1.6229x over previous
import functools

import jax
import jax.numpy as jnp
from jax import lax
from jax.experimental import pallas as pl
from jax.experimental.pallas import tpu as pltpu
from jax.experimental.pallas import tpu_sc as plsc

VOCAB = 1000000
EMBED_DIM = 32
BATCH = 16384
HIST_LEN = 50

NC = 2
NS = 16
NW = NC * NS
R_PER_W = BATCH // NW

RCHUNK = 16
NCHUNKS = R_PER_W // RCHUNK
NPAIRS = NCHUNKS // 2

_mesh = plsc.VectorSubcoreMesh(core_axis_name="c", subcore_axis_name="s")


@functools.partial(
    pl.kernel,
    out_type=jax.ShapeDtypeStruct((BATCH, HIST_LEN, EMBED_DIM), jnp.float32),
    mesh=_mesh,
    scratch_types=[
        pltpu.VMEM((R_PER_W, HIST_LEN), jnp.int32),
        pltpu.VMEM((RCHUNK, HIST_LEN, EMBED_DIM), jnp.float32),
        pltpu.VMEM((RCHUNK, HIST_LEN, EMBED_DIM), jnp.float32),
        pltpu.SemaphoreType.DMA,
        pltpu.SemaphoreType.DMA,
        pltpu.SemaphoreType.DMA,
        pltpu.SemaphoreType.DMA,
        pltpu.SemaphoreType.DMA,
    ],
    compiler_params=pltpu.CompilerParams(use_tc_tiling_on_sc=False),
)
def _emb_gather(title_hbm, table_hbm, out_hbm, idx_all, rows0, rows1,
                isem, gsem0, gsem1, wsem0, wsem1):
    wid = lax.axis_index("s") * NC + lax.axis_index("c")
    base = pl.multiple_of(wid * R_PER_W, R_PER_W)

    pltpu.async_copy(title_hbm.at[pl.ds(base, R_PER_W)], idx_all, isem).wait()

    rows = (rows0, rows1)
    gsem = (gsem0, gsem1)
    wsem = (wsem0, wsem1)

    def fire_gathers(c, slot):
        for j in range(RCHUNK):
            pltpu.async_copy(
                table_hbm.at[idx_all.at[c * RCHUNK + j]],
                rows[slot].at[j],
                gsem[slot],
            )

    def drain_gathers(slot):
        pltpu.make_async_copy(
            out_hbm.at[pl.ds(0, RCHUNK)], rows[slot], gsem[slot]
        ).wait()

    def fire_write(c, slot):
        pltpu.async_copy(
            rows[slot], out_hbm.at[pl.ds(base + c * RCHUNK, RCHUNK)], wsem[slot]
        )

    def drain_write(slot):
        pltpu.make_async_copy(
            rows[slot], out_hbm.at[pl.ds(0, RCHUNK)], wsem[slot]
        ).wait()

    fire_gathers(0, 0)

    def pair_body(p, carry):
        c0 = p * 2
        c1 = c0 + 1

        @pl.when(p > 0)
        def _():
            drain_write(1)

        fire_gathers(c1, 1)

        drain_gathers(0)
        fire_write(c0, 0)

        @pl.when(p + 1 < NPAIRS)
        def _():
            drain_write(0)
            fire_gathers(c0 + 2, 0)

        drain_gathers(1)
        fire_write(c1, 1)
        return carry

    lax.fori_loop(0, NPAIRS, pair_body, 0)
    drain_write(0)
    drain_write(1)


def kernel(title, table):
    return _emb_gather(title.astype(jnp.int32), table)
